# speedup vs baseline: 2.3770x; 2.3770x over previous
import functools

import jax
import jax.numpy as jnp
from jax import lax
from jax.experimental import pallas as pl
from jax.experimental.pallas import tpu as pltpu
from jax.experimental.pallas import tpu_sc as plsc

E = 8
K = 2
D = 768
F = 3072
T = 2048
A = T * K
M = 128
NP = 5120
NB = NP // M

NC = 2
NS = 16
NW = NC * NS


def _router_body(x_ref, wr_ref, idx_ref, gate_ref):
    xb = x_ref[...]
    logits = jnp.dot(xb, wr_ref[...], preferred_element_type=jnp.float32)
    m = jnp.max(logits, axis=1, keepdims=True)
    ex = jnp.exp(logits - m)
    probs = ex / jnp.sum(ex, axis=1, keepdims=True)
    li = lax.broadcasted_iota(jnp.int32, probs.shape, 1)
    m1 = jnp.max(probs, axis=1, keepdims=True)
    i1 = jnp.min(jnp.where(probs == m1, li, E), axis=1, keepdims=True)
    masked = jnp.where(li == i1, -1.0, probs)
    m2 = jnp.max(masked, axis=1, keepdims=True)
    i2 = jnp.min(jnp.where(masked == m2, li, E), axis=1, keepdims=True)
    bs = xb.shape[0]
    zi = jnp.zeros((bs, E - 2), jnp.int32)
    zf = jnp.zeros((bs, E - 2), jnp.float32)
    idx_ref[...] = jnp.concatenate([i1, i2, zi], axis=1)
    gate_ref[...] = jnp.concatenate([m1, m2, zf], axis=1)


def _router(x2d, wr):
    bs = 256
    return pl.pallas_call(
        _router_body,
        grid=(T // bs,),
        in_specs=[
            pl.BlockSpec((bs, D), lambda i: (i, 0)),
            pl.BlockSpec((D, E), lambda i: (0, 0)),
        ],
        out_specs=[
            pl.BlockSpec((bs, E), lambda i: (i, 0)),
            pl.BlockSpec((bs, E), lambda i: (i, 0)),
        ],
        out_shape=[
            jax.ShapeDtypeStruct((T, E), jnp.int32),
            jax.ShapeDtypeStruct((T, E), jnp.float32),
        ],
    )(x2d, wr)


def _sc_gather(tok_for_slot, x2d):
    per_w = NP // NW
    ch = per_w // 2

    mesh = plsc.VectorSubcoreMesh(core_axis_name="c", subcore_axis_name="s")

    @functools.partial(
        pl.kernel,
        mesh=mesh,
        out_type=jax.ShapeDtypeStruct((NP, D), jnp.float32),
        scratch_types=[
            pltpu.VMEM((ch,), jnp.int32),
            pltpu.VMEM((ch, D), jnp.float32),
            pltpu.SemaphoreType.DMA,
        ],
    )
    def k(tok_hbm, x_hbm, xs_hbm, idx_v, rows_v, sem):
        wid = lax.axis_index("s") * NC + lax.axis_index("c")
        for c in range(per_w // ch):
            base = wid * per_w + c * ch
            pltpu.sync_copy(tok_hbm.at[pl.ds(base, ch)], idx_v)
            pltpu.async_copy(x_hbm.at[idx_v], rows_v, sem).wait()
            pltpu.sync_copy(rows_v, xs_hbm.at[pl.ds(base, ch)])

    return k(tok_for_slot, x2d)


def _mlp_body(be_ref, xs_ref, w1_ref, b1_ref, w2_ref, b2_ref, g_ref, out_ref):
    h = jnp.dot(xs_ref[...], w1_ref[0], preferred_element_type=jnp.float32)
    h = h + b1_ref[0]
    h = 0.5 * h * (1.0 + lax.erf(h * (2.0 ** -0.5)))
    y = jnp.dot(h, w2_ref[0], preferred_element_type=jnp.float32)
    y = y + b2_ref[0]
    out_ref[...] = y * g_ref[...]


def _grouped_mlp(be, xs, w1, b1r, w2, b2r, gate2d):
    grid_spec = pltpu.PrefetchScalarGridSpec(
        num_scalar_prefetch=1,
        grid=(NB,),
        in_specs=[
            pl.BlockSpec((M, D), lambda i, be_ref: (i, 0)),
            pl.BlockSpec((1, D, F), lambda i, be_ref: (be_ref[i], 0, 0)),
            pl.BlockSpec((1, 1, F), lambda i, be_ref: (be_ref[i], 0, 0)),
            pl.BlockSpec((1, F, D), lambda i, be_ref: (be_ref[i], 0, 0)),
            pl.BlockSpec((1, 1, D), lambda i, be_ref: (be_ref[i], 0, 0)),
            pl.BlockSpec((M, 1), lambda i, be_ref: (i, 0)),
        ],
        out_specs=pl.BlockSpec((M, D), lambda i, be_ref: (i, 0)),
    )
    return pl.pallas_call(
        _mlp_body,
        grid_spec=grid_spec,
        out_shape=jax.ShapeDtypeStruct((NP, D), jnp.float32),
        compiler_params=pltpu.CompilerParams(
            dimension_semantics=("arbitrary",),
        ),
    )(be, xs, w1, b1r, w2, b2r, gate2d)


def _sc_combine(s0, s1, ysg):
    per_w = T // NW

    mesh = plsc.VectorSubcoreMesh(core_axis_name="c", subcore_axis_name="s")

    @functools.partial(
        pl.kernel,
        mesh=mesh,
        out_type=jax.ShapeDtypeStruct((T, D), jnp.float32),
        scratch_types=[
            pltpu.VMEM((per_w,), jnp.int32),
            pltpu.VMEM((per_w,), jnp.int32),
            pltpu.VMEM((per_w, D), jnp.float32),
            pltpu.VMEM((per_w, D), jnp.float32),
            pltpu.SemaphoreType.DMA,
        ],
    )
    def k(s0_hbm, s1_hbm, ys_hbm, out_hbm, i0_v, i1_v, r0, r1, sem):
        wid = lax.axis_index("s") * NC + lax.axis_index("c")
        base = wid * per_w
        pltpu.sync_copy(s0_hbm.at[pl.ds(base, per_w)], i0_v)
        pltpu.sync_copy(s1_hbm.at[pl.ds(base, per_w)], i1_v)
        cp0 = pltpu.async_copy(ys_hbm.at[i0_v], r0, sem)
        cp1 = pltpu.async_copy(ys_hbm.at[i1_v], r1, sem)
        cp0.wait()
        cp1.wait()

        def row_body(r, _):
            def col_body(c, _):
                sl = pl.ds(c * 16, 16)
                r0[r, sl] = r0[r, sl] + r1[r, sl]
                return 0

            return lax.fori_loop(0, D // 16, col_body, 0)

        lax.fori_loop(0, per_w, row_body, 0)
        pltpu.sync_copy(r0, out_hbm.at[pl.ds(base, per_w)])

    return k(s0, s1, ysg)


def _dispatch(idx2, gate2):
    eid = idx2.reshape(-1)
    g = gate2.reshape(-1)
    oh = (eid[:, None] == jnp.arange(E, dtype=jnp.int32)[None, :]).astype(
        jnp.int32)
    csum = jnp.cumsum(oh, axis=0)
    rank = jnp.take_along_axis(csum, eid[:, None], axis=1)[:, 0] - 1
    counts = csum[-1]
    padded = ((counts + (M - 1)) // M) * M
    offsets = jnp.concatenate(
        [jnp.zeros((1,), jnp.int32), jnp.cumsum(padded).astype(jnp.int32)])
    slot = jnp.take(offsets, eid) + rank
    tokid = (jnp.arange(A, dtype=jnp.int32) // K)
    tok_for_slot = jnp.zeros((NP,), jnp.int32).at[slot].set(tokid)
    gate_for_slot = jnp.zeros((NP,), jnp.float32).at[slot].set(g)
    q = jnp.arange(NB, dtype=jnp.int32) * M
    be = jnp.sum((offsets[1:][None, :] <= q[:, None]).astype(jnp.int32),
                 axis=1)
    be = jnp.minimum(be, E - 1).astype(jnp.int32)
    s01 = slot.reshape(T, K)
    return (tok_for_slot, gate_for_slot.reshape(NP, 1), be,
            s01[:, 0].astype(jnp.int32), s01[:, 1].astype(jnp.int32))


def kernel(x, Wr, W1, b1, W2, b2):
    B, S, _ = x.shape
    x2d = x.reshape(T, D)
    idx8, gate8 = _router(x2d, Wr)
    tok_for_slot, gate_for_slot, be, s0, s1 = _dispatch(
        idx8[:, :K], gate8[:, :K])
    xs = _sc_gather(tok_for_slot, x2d)
    b1r = b1.reshape(E, 1, F)
    b2r = b2.reshape(E, 1, D)
    ysg = _grouped_mlp(be, xs, W1, b1r, W2, b2r, gate_for_slot)
    out = _sc_combine(s0, s1, ysg)
    return out.reshape(B, S, D)


# speedup vs baseline: 3.1043x; 1.3060x over previous
import functools

import jax
import jax.numpy as jnp
from jax import lax
from jax.experimental import pallas as pl
from jax.experimental.pallas import tpu as pltpu
from jax.experimental.pallas import tpu_sc as plsc

E = 8
K = 2
D = 768
F = 3072
T = 2048
A = T * K
M = 128
NP = 5120
NB = NP // M

NC = 2
NS = 16
NW = NC * NS


def _router_body(x_ref, wr_ref, idx_ref, gate_ref):
    xb = x_ref[...]
    logits = jnp.dot(xb, wr_ref[...], preferred_element_type=jnp.float32)
    m = jnp.max(logits, axis=1, keepdims=True)
    ex = jnp.exp(logits - m)
    probs = ex / jnp.sum(ex, axis=1, keepdims=True)
    li = lax.broadcasted_iota(jnp.int32, probs.shape, 1)
    m1 = jnp.max(probs, axis=1, keepdims=True)
    i1 = jnp.min(jnp.where(probs == m1, li, E), axis=1, keepdims=True)
    masked = jnp.where(li == i1, -1.0, probs)
    m2 = jnp.max(masked, axis=1, keepdims=True)
    i2 = jnp.min(jnp.where(masked == m2, li, E), axis=1, keepdims=True)
    bs = xb.shape[0]
    zi = jnp.zeros((bs, E - 2), jnp.int32)
    zf = jnp.zeros((bs, E - 2), jnp.float32)
    idx_ref[...] = jnp.concatenate([i1, i2, zi], axis=1)
    gate_ref[...] = jnp.concatenate([m1, m2, zf], axis=1)


def _router(x2d, wr):
    bs = 256
    return pl.pallas_call(
        _router_body,
        grid=(T // bs,),
        in_specs=[
            pl.BlockSpec((bs, D), lambda i: (i, 0)),
            pl.BlockSpec((D, E), lambda i: (0, 0)),
        ],
        out_specs=[
            pl.BlockSpec((bs, E), lambda i: (i, 0)),
            pl.BlockSpec((bs, E), lambda i: (i, 0)),
        ],
        out_shape=[
            jax.ShapeDtypeStruct((T, E), jnp.int32),
            jax.ShapeDtypeStruct((T, E), jnp.float32),
        ],
    )(x2d, wr)


def _sc_dispatch(x2d, s0, s1):
    tw = T // NW

    mesh = plsc.VectorSubcoreMesh(core_axis_name="c", subcore_axis_name="s")

    @functools.partial(
        pl.kernel,
        mesh=mesh,
        out_type=jax.ShapeDtypeStruct((NP, D), jnp.float32),
        scratch_types=[
            pltpu.VMEM((tw, D), jnp.float32),
            pltpu.VMEM((tw,), jnp.int32),
            pltpu.VMEM((tw,), jnp.int32),
            pltpu.SemaphoreType.DMA,
        ],
    )
    def k(x_hbm, s0_hbm, s1_hbm, xs_hbm, rows_v, i0_v, i1_v, sem):
        wid = lax.axis_index("s") * NC + lax.axis_index("c")
        base = wid * tw
        pltpu.sync_copy(x_hbm.at[pl.ds(base, tw)], rows_v)
        pltpu.sync_copy(s0_hbm.at[pl.ds(base, tw)], i0_v)
        pltpu.sync_copy(s1_hbm.at[pl.ds(base, tw)], i1_v)
        cp0 = pltpu.async_copy(rows_v, xs_hbm.at[i0_v], sem)
        cp1 = pltpu.async_copy(rows_v, xs_hbm.at[i1_v], sem)
        cp0.wait()
        cp1.wait()

    return k(x2d, s0, s1)


def _mlp_body(be_ref, xs_ref, w1_ref, b1_ref, w2_ref, b2_ref, out_ref):
    h = jnp.dot(xs_ref[...], w1_ref[0], preferred_element_type=jnp.float32)
    h = h + b1_ref[0]
    h = 0.5 * h * (1.0 + lax.erf(h * (2.0 ** -0.5)))
    y = jnp.dot(h, w2_ref[0], preferred_element_type=jnp.float32)
    out_ref[...] = y + b2_ref[0]


def _grouped_mlp(be, xs, w1, b1r, w2, b2r):
    grid_spec = pltpu.PrefetchScalarGridSpec(
        num_scalar_prefetch=1,
        grid=(NB,),
        in_specs=[
            pl.BlockSpec((M, D), lambda i, be_ref: (i, 0)),
            pl.BlockSpec((1, D, F), lambda i, be_ref: (be_ref[i], 0, 0)),
            pl.BlockSpec((1, 1, F), lambda i, be_ref: (be_ref[i], 0, 0)),
            pl.BlockSpec((1, F, D), lambda i, be_ref: (be_ref[i], 0, 0)),
            pl.BlockSpec((1, 1, D), lambda i, be_ref: (be_ref[i], 0, 0)),
        ],
        out_specs=pl.BlockSpec((M, D), lambda i, be_ref: (i, 0)),
    )
    return pl.pallas_call(
        _mlp_body,
        grid_spec=grid_spec,
        out_shape=jax.ShapeDtypeStruct((NP, D), jnp.float32),
        compiler_params=pltpu.CompilerParams(
            dimension_semantics=("arbitrary",),
        ),
    )(be, xs, w1, b1r, w2, b2r)


def _sc_combine(s0, s1, g0x, g1x, ys):
    per_w = T // NW

    mesh = plsc.VectorSubcoreMesh(core_axis_name="c", subcore_axis_name="s")

    @functools.partial(
        pl.kernel,
        mesh=mesh,
        out_type=jax.ShapeDtypeStruct((T, D), jnp.float32),
        scratch_types=[
            pltpu.VMEM((per_w,), jnp.int32),
            pltpu.VMEM((per_w,), jnp.int32),
            pltpu.VMEM((per_w, 16), jnp.float32),
            pltpu.VMEM((per_w, 16), jnp.float32),
            pltpu.VMEM((per_w, D), jnp.float32),
            pltpu.VMEM((per_w, D), jnp.float32),
            pltpu.SemaphoreType.DMA,
        ],
    )
    def k(s0_hbm, s1_hbm, g0_hbm, g1_hbm, ys_hbm, out_hbm,
          i0_v, i1_v, gv0, gv1, r0, r1, sem):
        wid = lax.axis_index("s") * NC + lax.axis_index("c")
        base = wid * per_w
        pltpu.sync_copy(s0_hbm.at[pl.ds(base, per_w)], i0_v)
        pltpu.sync_copy(s1_hbm.at[pl.ds(base, per_w)], i1_v)
        pltpu.sync_copy(g0_hbm.at[pl.ds(base, per_w)], gv0)
        pltpu.sync_copy(g1_hbm.at[pl.ds(base, per_w)], gv1)
        cp0 = pltpu.async_copy(ys_hbm.at[i0_v], r0, sem)
        cp1 = pltpu.async_copy(ys_hbm.at[i1_v], r1, sem)
        cp0.wait()
        cp1.wait()

        def row_body(r, _):
            ga = gv0[r, pl.ds(0, 16)]
            gb = gv1[r, pl.ds(0, 16)]

            def col_body(c, _):
                sl = pl.ds(c * 16, 16)
                r0[r, sl] = r0[r, sl] * ga + r1[r, sl] * gb
                return 0

            return lax.fori_loop(0, D // 16, col_body, 0)

        lax.fori_loop(0, per_w, row_body, 0)
        pltpu.sync_copy(r0, out_hbm.at[pl.ds(base, per_w)])

    return k(s0, s1, g0x, g1x, ys)


def _dispatch(idx2, gate2):
    eid = idx2.reshape(-1)
    g = gate2.reshape(-1)
    oh = (eid[:, None] == jnp.arange(E, dtype=jnp.int32)[None, :]).astype(
        jnp.int32)
    csum = jnp.cumsum(oh, axis=0)
    rank = jnp.take_along_axis(csum, eid[:, None], axis=1)[:, 0] - 1
    counts = csum[-1]
    padded = ((counts + (M - 1)) // M) * M
    offsets = jnp.concatenate(
        [jnp.zeros((1,), jnp.int32), jnp.cumsum(padded).astype(jnp.int32)])
    slot = jnp.take(offsets, eid) + rank
    q = jnp.arange(NB, dtype=jnp.int32) * M
    be = jnp.sum((offsets[1:][None, :] <= q[:, None]).astype(jnp.int32),
                 axis=1)
    be = jnp.minimum(be, E - 1).astype(jnp.int32)
    s01 = slot.reshape(T, K)
    g01 = g.reshape(T, K)
    g0x = jnp.broadcast_to(g01[:, 0:1], (T, 16))
    g1x = jnp.broadcast_to(g01[:, 1:2], (T, 16))
    return (be, s01[:, 0].astype(jnp.int32), s01[:, 1].astype(jnp.int32),
            g0x, g1x)


def kernel(x, Wr, W1, b1, W2, b2):
    B, S, _ = x.shape
    x2d = x.reshape(T, D)
    idx8, gate8 = _router(x2d, Wr)
    be, s0, s1, g0x, g1x = _dispatch(idx8[:, :K], gate8[:, :K])
    xs = _sc_dispatch(x2d, s0, s1)
    b1r = b1.reshape(E, 1, F)
    b2r = b2.reshape(E, 1, D)
    ys = _grouped_mlp(be, xs, W1, b1r, W2, b2r)
    out = _sc_combine(s0, s1, g0x, g1x, ys)
    return out.reshape(B, S, D)
